# baseline (speedup 1.0000x reference)
import functools

import jax
import jax.numpy as jnp
from jax import lax
from jax.experimental import pallas as pl
from jax.experimental.pallas import tpu as pltpu
from jax.experimental.pallas import tpu_sc as plsc

_NC = 2
_NS = 16
_NW = _NC * _NS


def _relayout_body(d_mf, d_mlp, *refs):
    tmf_u, tmf_i, tmlp_u, tmlp_i = refs[:4]
    outs = refs[4:-1]
    sem = refs[-1]
    cps = []
    o = 0
    for t, d in ((tmf_u, d_mf), (tmf_i, d_mf), (tmlp_u, d_mlp), (tmlp_i, d_mlp)):
        for j in range(d):
            cps.append(pltpu.make_async_copy(t.at[j], outs[o], sem))
            o += 1
    for c in cps:
        c.start()
    for c in cps:
        c.wait()


def _tc_relayout(tmf_u, tmf_i, tmlp_u, tmlp_i):
    v = tmf_u.shape[1]
    d_mf = tmf_u.shape[0]
    d_mlp = tmlp_u.shape[0]
    n_out = 2 * d_mf + 2 * d_mlp
    return pl.pallas_call(
        functools.partial(_relayout_body, d_mf, d_mlp),
        in_specs=[pl.BlockSpec(memory_space=pl.ANY)] * 4,
        out_specs=[pl.BlockSpec(memory_space=pl.ANY)] * n_out,
        out_shape=[jax.ShapeDtypeStruct((v,), jnp.float32)] * n_out,
        scratch_shapes=[pltpu.SemaphoreType.DMA],
    )(tmf_u, tmf_i, tmlp_u, tmlp_i)


def _sc_gather_body(bpw, d_mf, d_mlp, *refs):
    uid_h, iid_h = refs[0], refs[1]
    p = 2
    umf_h = refs[p:p + d_mf]; p += d_mf
    imf_h = refs[p:p + d_mf]; p += d_mf
    umlp_h = refs[p:p + d_mlp]; p += d_mlp
    imlp_h = refs[p:p + d_mlp]; p += d_mlp
    ub_h, ib_h = refs[p], refs[p + 1]; p += 2
    gmf_o, umlp_o, imlp_o, bias_o = refs[p:p + 4]; p += 4
    (uid_v, iid_v, umf_v, imf_v, umlp_v, imlp_v, ub_v, ib_v, gmf_v, bias_v,
     sem) = refs[p:]

    wid = lax.axis_index("s") * _NC + lax.axis_index("c")
    base = wid * bpw
    pltpu.sync_copy(uid_h.at[pl.ds(base, bpw)], uid_v)
    pltpu.sync_copy(iid_h.at[pl.ds(base, bpw)], iid_v)
    cps = []
    for j in range(d_mf):
        cps.append(pltpu.async_copy(umf_h[j].at[uid_v], umf_v.at[j], sem))
        cps.append(pltpu.async_copy(imf_h[j].at[iid_v], imf_v.at[j], sem))
    for j in range(d_mlp):
        cps.append(pltpu.async_copy(umlp_h[j].at[uid_v], umlp_v.at[j], sem))
        cps.append(pltpu.async_copy(imlp_h[j].at[iid_v], imlp_v.at[j], sem))
    cps.append(pltpu.async_copy(ub_h.at[uid_v], ub_v, sem))
    cps.append(pltpu.async_copy(ib_h.at[iid_v], ib_v, sem))
    for c in cps:
        c.wait()

    nk = bpw // 16
    for j in range(d_mf):
        def gmf_step(k, _, j=j):
            s = pl.ds(k * 16, 16)
            gmf_v[j, s] = umf_v[j, s] * imf_v[j, s]
            return 0
        lax.fori_loop(0, nk, gmf_step, 0)

    def bias_step(k, _):
        s = pl.ds(k * 16, 16)
        bias_v[s] = ub_v[s] + ib_v[s]
        return 0
    lax.fori_loop(0, nk, bias_step, 0)

    pltpu.sync_copy(gmf_v, gmf_o.at[:, pl.ds(base, bpw)])
    pltpu.sync_copy(umlp_v, umlp_o.at[:, pl.ds(base, bpw)])
    pltpu.sync_copy(imlp_v, imlp_o.at[:, pl.ds(base, bpw)])
    pltpu.sync_copy(bias_v, bias_o.at[pl.ds(base, bpw)])


def _sc_gather(user_ids, item_ids, planes, ub, ib, d_mf, d_mlp):
    b = user_ids.shape[0]
    bpw = b // _NW
    f32 = jnp.float32
    mesh = plsc.VectorSubcoreMesh(core_axis_name="c", subcore_axis_name="s")
    k = pl.kernel(
        functools.partial(_sc_gather_body, bpw, d_mf, d_mlp),
        out_type=[
            jax.ShapeDtypeStruct((d_mf, b), f32),
            jax.ShapeDtypeStruct((d_mlp, b), f32),
            jax.ShapeDtypeStruct((d_mlp, b), f32),
            jax.ShapeDtypeStruct((b,), f32),
        ],
        mesh=mesh,
        compiler_params=pltpu.CompilerParams(use_tc_tiling_on_sc=False),
        scratch_types=[
            pltpu.VMEM((bpw,), jnp.int32),
            pltpu.VMEM((bpw,), jnp.int32),
            pltpu.VMEM((d_mf, bpw), f32),
            pltpu.VMEM((d_mf, bpw), f32),
            pltpu.VMEM((d_mlp, bpw), f32),
            pltpu.VMEM((d_mlp, bpw), f32),
            pltpu.VMEM((bpw,), f32),
            pltpu.VMEM((bpw,), f32),
            pltpu.VMEM((d_mf, bpw), f32),
            pltpu.VMEM((bpw,), f32),
            pltpu.SemaphoreType.DMA,
        ],
    )
    return k(user_ids, item_ids, *planes, ub.reshape(-1), ib.reshape(-1))


def _tc_mlp_body(gmf, umlp, imlp, bias, w1, b1, w2, b2, w3, b3, wo, bo, out):
    d_mlp = umlp.shape[0]
    d_mf = gmf.shape[0]
    dn = (((1,), (0,)), ((), ()))
    h = lax.dot_general(w1[:, :d_mlp], umlp[...], dn,
                        preferred_element_type=jnp.float32)
    h += lax.dot_general(w1[:, d_mlp:], imlp[...], dn,
                         preferred_element_type=jnp.float32)
    h = jnp.maximum(h + b1[...], 0.0)
    h = jnp.maximum(
        lax.dot_general(w2[...], h, dn, preferred_element_type=jnp.float32)
        + b2[...], 0.0)
    h = jnp.maximum(
        lax.dot_general(w3[...], h, dn, preferred_element_type=jnp.float32)
        + b3[...], 0.0)
    o = lax.dot_general(wo[:, :d_mf], gmf[...], dn,
                        preferred_element_type=jnp.float32)
    o += lax.dot_general(wo[:, d_mf:], h, dn,
                         preferred_element_type=jnp.float32)
    out[...] = o + bo[...] + bias[...]


def _tc_mlp(gmf_t, umlp_t, imlp_t, bias_t, W1, b1, W2, b2, W3, b3, W_out, b_out):
    b = bias_t.shape[1]
    cb = 4096
    grid = (b // cb,)
    d_mf = gmf_t.shape[0]
    d_mlp = umlp_t.shape[0]

    def col_spec(d):
        return pl.BlockSpec((d, cb), lambda i: (0, i))

    def full_spec(a):
        return pl.BlockSpec(a.shape, lambda i: (0,) * a.ndim)

    return pl.pallas_call(
        _tc_mlp_body,
        grid=grid,
        in_specs=[
            col_spec(d_mf), col_spec(d_mlp), col_spec(d_mlp), col_spec(1),
            full_spec(W1), full_spec(b1), full_spec(W2), full_spec(b2),
            full_spec(W3), full_spec(b3), full_spec(W_out), full_spec(b_out),
        ],
        out_specs=col_spec(1),
        out_shape=jax.ShapeDtypeStruct((1, b), jnp.float32),
    )(gmf_t, umlp_t, imlp_t, bias_t, W1, b1, W2, b2, W3, b3, W_out, b_out)


def kernel(user_ids, item_ids, user_mf_emb, item_mf_emb, user_mlp_emb,
           item_mlp_emb, user_bias_emb, item_bias_emb, W1, b1, W2, b2, W3, b3,
           W_out, b_out):
    d_mf = user_mf_emb.shape[1]
    d_mlp = user_mlp_emb.shape[1]
    planes = _tc_relayout(user_mf_emb.T, item_mf_emb.T,
                          user_mlp_emb.T, item_mlp_emb.T)
    gmf_t, umlp_t, imlp_t, bias_t = _sc_gather(
        user_ids, item_ids, planes, user_bias_emb, item_bias_emb, d_mf, d_mlp)
    out_t = _tc_mlp(gmf_t, umlp_t, imlp_t, bias_t.reshape(1, -1),
                    W1, b1.reshape(-1, 1), W2, b2.reshape(-1, 1),
                    W3, b3.reshape(-1, 1), W_out, b_out.reshape(-1, 1))
    return out_t.reshape(-1, 1)


# speedup vs baseline: 23.5601x; 23.5601x over previous
import functools

import jax
import jax.numpy as jnp
from jax import lax
from jax.experimental import pallas as pl
from jax.experimental.pallas import tpu as pltpu
from jax.experimental.pallas import tpu_sc as plsc

_NC = 2
_NS = 16
_NW = _NC * _NS


def _tile_gather_body(bpw, uid_h, iid_h, tmf_u, tmf_i, tmlp_u, tmlp_i,
                      gmf_o, umlp_o, imlp_o,
                      uid_v, iid_v, bmf_u, bmf_i, bmlp_u, bmlp_i,
                      cmf_u, cmf_i, cmlp_u, cmlp_i, sem):
    wid = lax.axis_index("s") * _NC + lax.axis_index("c")
    base = wid * bpw
    pair = pl.multiple_of((wid // 2) * 1024, 1024)
    half = (wid % 2) * bpw
    pltpu.sync_copy(uid_h.at[pl.ds(pair, 1024)], uid_v)
    pltpu.sync_copy(iid_h.at[pl.ds(pair, 1024)], iid_v)

    lane = lax.broadcasted_iota(jnp.int32, (16,), 0)
    j8 = lane & 7
    zero = jnp.full((16,), 0, jnp.int32)

    for ch in range(bpw // 128):
        def group(gg, carry, ch=ch):
            goff = half + ch * 128 + gg * 16
            uvec = uid_v[pl.ds(goff, 16)]
            ivec = iid_v[pl.ds(goff, 16)]
            kbase = gg * 16

            for sub in range(2):
                cps = []
                for k in range(8):
                    lk = sub * 8 + k
                    us = pl.multiple_of((uvec[lk] >> 7) * 128, 128)
                    is_ = pl.multiple_of((ivec[lk] >> 7) * 128, 128)
                    cps.append(pltpu.async_copy(
                        tmf_u.at[:, pl.ds(us, 128)], bmf_u.at[k], sem))
                    cps.append(pltpu.async_copy(
                        tmf_i.at[:, pl.ds(is_, 128)], bmf_i.at[k], sem))
                    cps.append(pltpu.async_copy(
                        tmlp_u.at[:, pl.ds(us, 128)], bmlp_u.at[k], sem))
                    cps.append(pltpu.async_copy(
                        tmlp_i.at[:, pl.ds(is_, 128)], bmlp_i.at[k], sem))
                for c in cps:
                    c.wait()
                for p in range(4):
                    k0, k1 = 2 * p, 2 * p + 1
                    bsel = jnp.where(lane < 8, k0, k1)
                    kvec = kbase + sub * 8 + bsel
                    lu = jnp.where(lane < 8, uvec[sub * 8 + k0] & 127,
                                   uvec[sub * 8 + k1] & 127)
                    li = jnp.where(lane < 8, ivec[sub * 8 + k0] & 127,
                                   ivec[sub * 8 + k1] & 127)
                    vu = plsc.load_gather(bmf_u, [bsel, j8, lu])
                    vi = plsc.load_gather(bmf_i, [bsel, j8, li])
                    plsc.store_scatter(cmf_u, [j8, kvec], vu)
                    plsc.store_scatter(cmf_i, [j8, kvec], vi)
                for k in range(8):
                    lk = sub * 8 + k
                    lu = zero + (uvec[lk] & 127)
                    li = zero + (ivec[lk] & 127)
                    kv = zero + (kbase + lk)
                    ksel = zero + k
                    for hh in range(2):
                        jv = lane + hh * 16
                        vu = plsc.load_gather(bmlp_u, [ksel, jv, lu])
                        vi = plsc.load_gather(bmlp_i, [ksel, jv, li])
                        plsc.store_scatter(cmlp_u, [jv, kv], vu)
                        plsc.store_scatter(cmlp_i, [jv, kv], vi)
            return carry

        lax.fori_loop(0, 8, group, 0)
        for j in range(8):
            for q in range(8):
                s = pl.ds(q * 16, 16)
                cmf_u[j, s] = cmf_u[j, s] * cmf_i[j, s]
        off = base + ch * 128
        pltpu.sync_copy(cmf_u, gmf_o.at[:, pl.ds(off, 128)])
        pltpu.sync_copy(cmlp_u, umlp_o.at[:, pl.ds(off, 128)])
        pltpu.sync_copy(cmlp_i, imlp_o.at[:, pl.ds(off, 128)])


def _tile_gather(uids, iids, tmf_u, tmf_i, tmlp_u, tmlp_i):
    b = uids.shape[0]
    bpw = b // _NW
    d_mf = tmf_u.shape[0]
    d_mlp = tmlp_u.shape[0]
    f32 = jnp.float32
    mesh = plsc.VectorSubcoreMesh(core_axis_name="c", subcore_axis_name="s")
    k = pl.kernel(
        functools.partial(_tile_gather_body, bpw),
        out_type=[
            jax.ShapeDtypeStruct((d_mf, b), f32),
            jax.ShapeDtypeStruct((d_mlp, b), f32),
            jax.ShapeDtypeStruct((d_mlp, b), f32),
        ],
        mesh=mesh,
        compiler_params=pltpu.CompilerParams(
            use_tc_tiling_on_sc=True, needs_layout_passes=False),
        scratch_types=[
            pltpu.VMEM((1024,), jnp.int32),
            pltpu.VMEM((1024,), jnp.int32),
            pltpu.VMEM((8, d_mf, 128), f32),
            pltpu.VMEM((8, d_mf, 128), f32),
            pltpu.VMEM((8, d_mlp, 128), f32),
            pltpu.VMEM((8, d_mlp, 128), f32),
            pltpu.VMEM((d_mf, 128), f32),
            pltpu.VMEM((d_mf, 128), f32),
            pltpu.VMEM((d_mlp, 128), f32),
            pltpu.VMEM((d_mlp, 128), f32),
            pltpu.SemaphoreType.DMA,
        ],
    )
    return k(uids, iids, tmf_u, tmf_i, tmlp_u, tmlp_i)


def _bias_body(bpw, uid_h, iid_h, ub_h, ib_h, bias_o,
               uid_v, iid_v, ub_v, ib_v, sem):
    wid = lax.axis_index("s") * _NC + lax.axis_index("c")
    base = wid * bpw
    pltpu.sync_copy(uid_h.at[pl.ds(base, bpw)], uid_v)
    pltpu.sync_copy(iid_h.at[pl.ds(base, bpw)], iid_v)
    c1 = pltpu.async_copy(ub_h.at[uid_v], ub_v, sem)
    c2 = pltpu.async_copy(ib_h.at[iid_v], ib_v, sem)
    c1.wait()
    c2.wait()
    for q in range(bpw // 16):
        s = pl.ds(q * 16, 16)
        ub_v[s] = ub_v[s] + ib_v[s]
    pltpu.sync_copy(ub_v, bias_o.at[pl.ds(base, bpw)])


def _sc_bias(uids, iids, ub, ib):
    b = uids.shape[0]
    bpw = b // _NW
    f32 = jnp.float32
    mesh = plsc.VectorSubcoreMesh(core_axis_name="c", subcore_axis_name="s")
    k = pl.kernel(
        functools.partial(_bias_body, bpw),
        out_type=[jax.ShapeDtypeStruct((b,), f32)],
        mesh=mesh,
        compiler_params=pltpu.CompilerParams(use_tc_tiling_on_sc=False),
        scratch_types=[
            pltpu.VMEM((bpw,), jnp.int32),
            pltpu.VMEM((bpw,), jnp.int32),
            pltpu.VMEM((bpw,), f32),
            pltpu.VMEM((bpw,), f32),
            pltpu.SemaphoreType.DMA,
        ],
    )
    return k(uids, iids, ub.reshape(-1), ib.reshape(-1))[0]


def _tc_mlp_body(gmf, umlp, imlp, bias, w1, b1, w2, b2, w3, b3, wo, bo, out):
    d_mlp = umlp.shape[0]
    d_mf = gmf.shape[0]
    dn = (((1,), (0,)), ((), ()))
    h = lax.dot_general(w1[:, :d_mlp], umlp[...], dn,
                        preferred_element_type=jnp.float32)
    h += lax.dot_general(w1[:, d_mlp:], imlp[...], dn,
                         preferred_element_type=jnp.float32)
    h = jnp.maximum(h + b1[...], 0.0)
    h = jnp.maximum(
        lax.dot_general(w2[...], h, dn, preferred_element_type=jnp.float32)
        + b2[...], 0.0)
    h = jnp.maximum(
        lax.dot_general(w3[...], h, dn, preferred_element_type=jnp.float32)
        + b3[...], 0.0)
    o = lax.dot_general(wo[:, :d_mf], gmf[...], dn,
                        preferred_element_type=jnp.float32)
    o += lax.dot_general(wo[:, d_mf:], h, dn,
                         preferred_element_type=jnp.float32)
    out[...] = o + bo[...] + bias[...]


def _tc_mlp(gmf_t, umlp_t, imlp_t, bias_t, W1, b1, W2, b2, W3, b3, W_out, b_out):
    b = bias_t.shape[1]
    cb = 4096
    grid = (b // cb,)
    d_mf = gmf_t.shape[0]
    d_mlp = umlp_t.shape[0]

    def col_spec(d):
        return pl.BlockSpec((d, cb), lambda i: (0, i))

    def full_spec(a):
        return pl.BlockSpec(a.shape, lambda i: (0,) * a.ndim)

    return pl.pallas_call(
        _tc_mlp_body,
        grid=grid,
        in_specs=[
            col_spec(d_mf), col_spec(d_mlp), col_spec(d_mlp), col_spec(1),
            full_spec(W1), full_spec(b1), full_spec(W2), full_spec(b2),
            full_spec(W3), full_spec(b3), full_spec(W_out), full_spec(b_out),
        ],
        out_specs=col_spec(1),
        out_shape=jax.ShapeDtypeStruct((1, b), jnp.float32),
    )(gmf_t, umlp_t, imlp_t, bias_t, W1, b1, W2, b2, W3, b3, W_out, b_out)


def kernel(user_ids, item_ids, user_mf_emb, item_mf_emb, user_mlp_emb,
           item_mlp_emb, user_bias_emb, item_bias_emb, W1, b1, W2, b2, W3, b3,
           W_out, b_out):
    gmf_t, umlp_t, imlp_t = _tile_gather(
        user_ids, item_ids, user_mf_emb.T, item_mf_emb.T,
        user_mlp_emb.T, item_mlp_emb.T)
    bias = _sc_bias(user_ids, item_ids, user_bias_emb, item_bias_emb)
    out_t = _tc_mlp(gmf_t, umlp_t, imlp_t, bias.reshape(1, -1),
                    W1, b1.reshape(-1, 1), W2, b2.reshape(-1, 1),
                    W3, b3.reshape(-1, 1), W_out, b_out.reshape(-1, 1))
    return out_t.reshape(-1, 1)
